# Initial kernel scaffold; baseline (speedup 1.0000x reference)
import jax
import jax.numpy as jnp
from jax.experimental import pallas as pl


def kernel(confidences, wealth):
    raise NotImplementedError("write your pallas kernel here")


# speedup vs baseline: 2.2300x; 2.2300x over previous
import functools

import jax
import jax.numpy as jnp
from jax import lax
from jax.experimental import pallas as pl
from jax.experimental.pallas import tpu as pltpu
from jax.experimental.pallas import tpu_sc as plsc

NUM_EXPERTS = 64
TOP_K = 2
BATCH = 4
SEQ = 8192
TOKENS = BATCH * SEQ

_INFO = plsc.get_sparse_core_info()
NC = _INFO.num_cores
NS = _INFO.num_subcores
LANES = _INFO.num_lanes
NW = NC * NS
TPW = TOKENS // NW
GROUPS = TPW // LANES

_mesh = plsc.VectorSubcoreMesh(core_axis_name="c", subcore_axis_name="s")


@functools.partial(
    pl.kernel,
    out_type=(
        jax.ShapeDtypeStruct((TOKENS * TOP_K,), jnp.int32),
        jax.ShapeDtypeStruct((TOKENS * TOP_K,), jnp.float32),
        jax.ShapeDtypeStruct((TOKENS * TOP_K,), jnp.float32),
    ),
    mesh=_mesh,
    compiler_params=pltpu.CompilerParams(needs_layout_passes=False),
    scratch_types=[
        pltpu.VMEM((TPW * NUM_EXPERTS,), jnp.float32),
        pltpu.VMEM((NUM_EXPERTS,), jnp.float32),
        pltpu.VMEM((TPW * TOP_K,), jnp.int32),
        pltpu.VMEM((TPW * TOP_K,), jnp.float32),
        pltpu.VMEM((TPW * TOP_K,), jnp.float32),
    ],
)
def _auction(conf_hbm, wealth_hbm, oidx_hbm, orw_hbm, opay_hbm,
             conf_v, wealth_v, oidx_v, orw_v, opay_v):
    wid = lax.axis_index("s") * NC + lax.axis_index("c")
    base = wid * TPW
    pltpu.sync_copy(conf_hbm.at[pl.ds(base * NUM_EXPERTS, TPW * NUM_EXPERTS)],
                    conf_v)
    pltpu.sync_copy(wealth_hbm, wealth_v)

    iota = lax.iota(jnp.int32, LANES)
    zeros = jnp.zeros((LANES,), jnp.int32)
    neg_inf = jnp.full((LANES,), -jnp.inf, jnp.float32)

    def group_body(g, carry):
        tokv = iota + g * LANES

        def expert_body(_, st):
            m1, m2, m3, i1, i2, ev, idxv = st
            col = plsc.load_gather(conf_v, [idxv])
            w = plsc.load_gather(wealth_v, [ev])
            b = col * w
            gt1 = b > m1
            gt2 = b > m2
            gt3 = b > m3
            nm3 = jnp.where(gt2, m2, jnp.where(gt3, b, m3))
            nm2 = jnp.where(gt1, m1, jnp.where(gt2, b, m2))
            ni2 = jnp.where(gt1, i1, jnp.where(gt2, ev, i2))
            nm1 = jnp.where(gt1, b, m1)
            ni1 = jnp.where(gt1, ev, i1)
            return nm1, nm2, nm3, ni1, ni2, ev + 1, idxv + 1

        m1, m2, m3, i1, i2, _, _ = lax.fori_loop(
            0, NUM_EXPERTS, expert_body,
            (neg_inf, neg_inf, neg_inf, zeros, zeros, zeros,
             tokv * NUM_EXPERTS))

        t = jnp.exp(m2 - m1)
        inv = 1.0 / (1.0 + t)
        pos = tokv * TOP_K
        plsc.store_scatter(oidx_v, [pos], i1)
        plsc.store_scatter(oidx_v, [pos + 1], i2)
        plsc.store_scatter(orw_v, [pos], inv)
        plsc.store_scatter(orw_v, [pos + 1], t * inv)
        plsc.store_scatter(opay_v, [pos], m3)
        plsc.store_scatter(opay_v, [pos + 1], m3)
        return carry

    lax.fori_loop(0, GROUPS, group_body, 0)

    pltpu.sync_copy(oidx_v, oidx_hbm.at[pl.ds(base * TOP_K, TPW * TOP_K)])
    pltpu.sync_copy(orw_v, orw_hbm.at[pl.ds(base * TOP_K, TPW * TOP_K)])
    pltpu.sync_copy(opay_v, opay_hbm.at[pl.ds(base * TOP_K, TPW * TOP_K)])


def kernel(confidences, wealth):
    conf = confidences.reshape(TOKENS * NUM_EXPERTS)
    oidx, orw, opay = _auction(conf, wealth)
    shape = (BATCH, SEQ, TOP_K)
    return (oidx.reshape(shape), orw.reshape(shape), opay.reshape(shape))


# speedup vs baseline: 2.3467x; 1.0523x over previous
import functools

import jax
import jax.numpy as jnp
from jax import lax
from jax.experimental import pallas as pl
from jax.experimental.pallas import tpu as pltpu
from jax.experimental.pallas import tpu_sc as plsc

NUM_EXPERTS = 64
TOP_K = 2
BATCH = 4
SEQ = 8192
TOKENS = BATCH * SEQ

_INFO = plsc.get_sparse_core_info()
NC = _INFO.num_cores
NS = _INFO.num_subcores
LANES = _INFO.num_lanes
NW = NC * NS
TPW = TOKENS // NW
GROUPS = TPW // LANES

_mesh = plsc.VectorSubcoreMesh(core_axis_name="c", subcore_axis_name="s")


@functools.partial(
    pl.kernel,
    out_type=(
        jax.ShapeDtypeStruct((TOKENS * TOP_K,), jnp.int32),
        jax.ShapeDtypeStruct((TOKENS * TOP_K,), jnp.float32),
        jax.ShapeDtypeStruct((TOKENS * TOP_K,), jnp.float32),
    ),
    mesh=_mesh,
    compiler_params=pltpu.CompilerParams(needs_layout_passes=False),
    scratch_types=[
        pltpu.VMEM((TPW * NUM_EXPERTS,), jnp.float32),
        pltpu.VMEM((NUM_EXPERTS,), jnp.float32),
        pltpu.VMEM((TPW * TOP_K,), jnp.int32),
        pltpu.VMEM((TPW * TOP_K,), jnp.float32),
        pltpu.VMEM((TPW * TOP_K,), jnp.float32),
    ],
)
def _auction(conf_hbm, wealth_hbm, oidx_hbm, orw_hbm, opay_hbm,
             conf_v, wealth_v, oidx_v, orw_v, opay_v):
    wid = lax.axis_index("s") * NC + lax.axis_index("c")
    base = wid * TPW
    pltpu.sync_copy(conf_hbm.at[pl.ds(base * NUM_EXPERTS, TPW * NUM_EXPERTS)],
                    conf_v)
    pltpu.sync_copy(wealth_hbm, wealth_v)

    iota = lax.iota(jnp.int32, LANES)
    zeros = jnp.zeros((LANES,), jnp.int32)
    neg_inf = jnp.full((LANES,), -jnp.inf, jnp.float32)

    def group_body(g, carry):
        tokv = iota + g * LANES
        idx0 = tokv * NUM_EXPERTS

        m1 = m2 = m3 = neg_inf
        i1 = i2 = zeros
        for e in range(NUM_EXPERTS):
            col = plsc.load_gather(conf_v, [idx0 + e])
            w = plsc.load_gather(wealth_v, [jnp.full((LANES,), e, jnp.int32)])
            b = col * w
            gt1 = b > m1
            gt2 = b > m2
            ev = jnp.full((LANES,), e, jnp.int32)
            nm3 = jnp.maximum(m3, jnp.minimum(m2, b))
            nm2 = jnp.maximum(m2, jnp.minimum(m1, b))
            ni2 = jnp.where(gt1, i1, jnp.where(gt2, ev, i2))
            nm1 = jnp.maximum(m1, b)
            ni1 = jnp.where(gt1, ev, i1)
            m1, m2, m3, i1, i2 = nm1, nm2, nm3, ni1, ni2

        t = jnp.exp(m2 - m1)
        inv = 1.0 / (1.0 + t)
        pos = tokv * TOP_K
        plsc.store_scatter(oidx_v, [pos], i1)
        plsc.store_scatter(oidx_v, [pos + 1], i2)
        plsc.store_scatter(orw_v, [pos], inv)
        plsc.store_scatter(orw_v, [pos + 1], t * inv)
        plsc.store_scatter(opay_v, [pos], m3)
        plsc.store_scatter(opay_v, [pos + 1], m3)
        return carry

    lax.fori_loop(0, GROUPS, group_body, 0)

    pltpu.sync_copy(oidx_v, oidx_hbm.at[pl.ds(base * TOP_K, TPW * TOP_K)])
    pltpu.sync_copy(orw_v, orw_hbm.at[pl.ds(base * TOP_K, TPW * TOP_K)])
    pltpu.sync_copy(opay_v, opay_hbm.at[pl.ds(base * TOP_K, TPW * TOP_K)])


def kernel(confidences, wealth):
    conf = confidences.reshape(TOKENS * NUM_EXPERTS)
    oidx, orw, opay = _auction(conf, wealth)
    shape = (BATCH, SEQ, TOP_K)
    return (oidx.reshape(shape), orw.reshape(shape), opay.reshape(shape))
